# baseline (speedup 1.0000x reference)
import functools

import jax
import jax.numpy as jnp
from jax import lax
from jax.experimental import pallas as pl
from jax.experimental.pallas import tpu as pltpu
from jax.experimental.pallas import tpu_sc as plsc

_EMBED = 32
_SEQ = 128
_POS_ROWS = 64
_LANES = 16
_GRP = 4
_CHUNK = _GRP * _SEQ * _EMBED


def _emb_kernel(patches_hbm, tok_hbm, pos_hbm, out_hbm,
                idx_v, g0, g1, g2, g3, o0, o1, posv, post_v, sem_g, sem_s):
    info = plsc.get_sparse_core_info()
    num_cores = info.num_cores
    num_workers = num_cores * info.num_subcores
    wid = lax.axis_index("s") * num_cores + lax.axis_index("c")

    batch = patches_hbm.shape[0]
    seqs_per_w = batch // num_workers
    n_groups = seqs_per_w // _GRP
    iota = lax.iota(jnp.int32, _LANES)

    pltpu.sync_copy(pos_hbm, posv)

    @plsc.parallel_loop(0, _EMBED)
    def post_body(e):
        ecol = jnp.full((_LANES,), e, jnp.int32)
        for s0 in range(_SEQ // _LANES):
            rows = jnp.minimum(iota + (s0 * _LANES), _POS_ROWS - 1)
            post_v[e, pl.ds(s0 * _LANES, _LANES)] = plsc.load_gather(
                posv, [rows, ecol])

    pltpu.sync_copy(patches_hbm.at[pl.ds(wid * seqs_per_w, seqs_per_w)], idx_v)

    gbufs = (g0, g1, g2, g3)
    obufs = (o0, o1)
    ahead = len(gbufs) - 1

    def issue_gathers(g, buf):
        for s in range(_GRP):
            pltpu.async_copy(tok_hbm.at[idx_v.at[g * _GRP + s]],
                             buf.at[pl.ds(s * _SEQ, _SEQ)], sem_g)

    def wait_bytes(buf, sem):
        src = (out_hbm.at[pl.ds(0, buf.shape[0])] if len(buf.shape) == 1
               else tok_hbm.at[pl.ds(0, buf.shape[0])])
        pltpu.make_async_copy(src, buf, sem).wait()

    def compute(gc, oc):
        @plsc.parallel_loop(0, _EMBED, unroll=2)
        def e_body(e):
            ecol = jnp.full((_LANES,), e, jnp.int32)
            ebase = e * _SEQ
            for s0 in range(_SEQ // _LANES):
                p = post_v[e, pl.ds(s0 * _LANES, _LANES)]
                for s in range(_GRP):
                    base = s * _SEQ + s0 * _LANES
                    v = plsc.load_gather(gc.at[pl.ds(base, _LANES)],
                                         [iota, ecol])
                    oc[pl.ds(ebase + (s * _SEQ * _EMBED + s0 * _LANES),
                             _LANES)] = v + p

    for k in range(ahead):
        issue_gathers(k, gbufs[k])

    def h_body(h, _):
        for b in range(len(gbufs)):
            g = h * len(gbufs) + b
            gc, oc = gbufs[b], obufs[b % 2]
            wait_bytes(gc, sem_g)
            @pl.when(g + ahead < n_groups)
            def _():
                issue_gathers(g + ahead, gbufs[(b + ahead) % len(gbufs)])

            @pl.when(g >= 2)
            def _():
                wait_bytes(oc, sem_s)
            compute(gc, oc)
            off = (wid * seqs_per_w + g * _GRP) * (_SEQ * _EMBED)
            pltpu.async_copy(oc, out_hbm.at[pl.ds(off, _CHUNK)], sem_s)
        return 0

    lax.fori_loop(0, n_groups // len(gbufs), h_body, 0)
    wait_bytes(o0, sem_s)
    wait_bytes(o1, sem_s)


def kernel(patches, token_table, pos_table):
    batch, seq = patches.shape
    vocab, embed = token_table.shape
    idx = patches.astype(jnp.int32)


    mesh = plsc.VectorSubcoreMesh(core_axis_name="c", subcore_axis_name="s")
    n_rows = batch * seq

    run = functools.partial(
        pl.kernel,
        out_type=jax.ShapeDtypeStruct((n_rows * embed,), jnp.float32),
        mesh=mesh,
        scratch_types=[
            pltpu.VMEM((batch // 32, seq), jnp.int32),
            pltpu.VMEM((_GRP * seq, embed), jnp.float32),
            pltpu.VMEM((_GRP * seq, embed), jnp.float32),
            pltpu.VMEM((_GRP * seq, embed), jnp.float32),
            pltpu.VMEM((_GRP * seq, embed), jnp.float32),
            pltpu.VMEM((_CHUNK,), jnp.float32),
            pltpu.VMEM((_CHUNK,), jnp.float32),
            pltpu.VMEM((_POS_ROWS, embed), jnp.float32),
            pltpu.VMEM((embed, seq), jnp.float32),
            pltpu.SemaphoreType.DMA,
            pltpu.SemaphoreType.DMA,
        ],
        compiler_params=pltpu.CompilerParams(use_tc_tiling_on_sc=False,
                                             needs_layout_passes=False),
    )(_emb_kernel)

    out = run(idx, token_table, pos_table)
    return out.reshape(batch, embed, seq).transpose(0, 2, 1)


# speedup vs baseline: 1.0289x; 1.0289x over previous
import functools

import jax
import jax.numpy as jnp
from jax import lax
from jax.experimental import pallas as pl
from jax.experimental.pallas import tpu as pltpu
from jax.experimental.pallas import tpu_sc as plsc

_EMBED = 32
_SEQ = 128
_POS_ROWS = 64
_LANES = 16
_GRP = 4
_CHUNK = _GRP * _SEQ * _EMBED


def _emb_kernel(patches_hbm, tok_hbm, pos_hbm, out_hbm,
                idx_v, g0, g1, o0, o1, posv, post_v, sem_g, sem_s):
    info = plsc.get_sparse_core_info()
    num_cores = info.num_cores
    num_workers = num_cores * info.num_subcores
    wid = lax.axis_index("s") * num_cores + lax.axis_index("c")

    batch = patches_hbm.shape[0]
    seqs_per_w = batch // num_workers
    n_groups = seqs_per_w // _GRP
    iota = lax.iota(jnp.int32, _LANES)

    pltpu.sync_copy(pos_hbm, posv)

    @plsc.parallel_loop(0, _EMBED)
    def post_body(e):
        ecol = jnp.full((_LANES,), e, jnp.int32)
        for s0 in range(_SEQ // _LANES):
            rows = jnp.minimum(iota + (s0 * _LANES), _POS_ROWS - 1)
            post_v[e, pl.ds(s0 * _LANES, _LANES)] = plsc.load_gather(
                posv, [rows, ecol])

    pltpu.sync_copy(patches_hbm.at[pl.ds(wid * seqs_per_w, seqs_per_w)], idx_v)

    gbufs = (g0, g1)
    obufs = (o0, o1)
    ahead = len(gbufs) - 1

    def issue_gathers(g, buf):
        for s in range(_GRP):
            pltpu.async_copy(tok_hbm.at[idx_v.at[g * _GRP + s]],
                             buf.at[pl.ds(s * _SEQ, _SEQ)], sem_g)

    def wait_bytes(buf, sem):
        src = (out_hbm.at[pl.ds(0, buf.shape[0])] if len(buf.shape) == 1
               else tok_hbm.at[pl.ds(0, buf.shape[0])])
        pltpu.make_async_copy(src, buf, sem).wait()

    def compute(gc, oc):
        @plsc.parallel_loop(0, _EMBED, unroll=4)
        def e_body(e):
            ecol = jnp.full((_LANES,), e, jnp.int32)
            ebase = e * _SEQ
            for s0 in range(_SEQ // _LANES):
                p = post_v[e, pl.ds(s0 * _LANES, _LANES)]
                for s in range(_GRP):
                    base = s * _SEQ + s0 * _LANES
                    v = plsc.load_gather(gc.at[pl.ds(base, _LANES)],
                                         [iota, ecol])
                    oc[pl.ds(ebase + (s * _SEQ * _EMBED + s0 * _LANES),
                             _LANES)] = v + p

    for k in range(ahead):
        issue_gathers(k, gbufs[k])

    def h_body(h, _):
        for b in range(len(gbufs)):
            g = h * len(gbufs) + b
            gc, oc = gbufs[b], obufs[b % 2]
            wait_bytes(gc, sem_g)
            @pl.when(g + ahead < n_groups)
            def _():
                issue_gathers(g + ahead, gbufs[(b + ahead) % len(gbufs)])

            @pl.when(g >= 2)
            def _():
                wait_bytes(oc, sem_s)
            compute(gc, oc)
            off = (wid * seqs_per_w + g * _GRP) * (_SEQ * _EMBED)
            pltpu.async_copy(oc, out_hbm.at[pl.ds(off, _CHUNK)], sem_s)
        return 0

    lax.fori_loop(0, n_groups // len(gbufs), h_body, 0)
    wait_bytes(o0, sem_s)
    wait_bytes(o1, sem_s)


def kernel(patches, token_table, pos_table):
    batch, seq = patches.shape
    vocab, embed = token_table.shape
    idx = patches.astype(jnp.int32)


    mesh = plsc.VectorSubcoreMesh(core_axis_name="c", subcore_axis_name="s")
    n_rows = batch * seq

    run = functools.partial(
        pl.kernel,
        out_type=jax.ShapeDtypeStruct((n_rows * embed,), jnp.float32),
        mesh=mesh,
        scratch_types=[
            pltpu.VMEM((batch // 32, seq), jnp.int32),
            pltpu.VMEM((_GRP * seq, embed), jnp.float32),
            pltpu.VMEM((_GRP * seq, embed), jnp.float32),
            pltpu.VMEM((_CHUNK,), jnp.float32),
            pltpu.VMEM((_CHUNK,), jnp.float32),
            pltpu.VMEM((_POS_ROWS, embed), jnp.float32),
            pltpu.VMEM((embed, seq), jnp.float32),
            pltpu.SemaphoreType.DMA,
            pltpu.SemaphoreType.DMA,
        ],
        compiler_params=pltpu.CompilerParams(use_tc_tiling_on_sc=False,
                                             needs_layout_passes=False),
    )(_emb_kernel)

    out = run(idx, token_table, pos_table)
    return out.reshape(batch, embed, seq).transpose(0, 2, 1)


# speedup vs baseline: 1.0469x; 1.0175x over previous
import functools

import jax
import jax.numpy as jnp
from jax import lax
from jax.experimental import pallas as pl
from jax.experimental.pallas import tpu as pltpu
from jax.experimental.pallas import tpu_sc as plsc

_EMBED = 32
_SEQ = 128
_POS_ROWS = 64
_LANES = 16
_GRP = 4
_CHUNK = _GRP * _SEQ * _EMBED


def _emb_kernel(patches_hbm, tok_hbm, pos_hbm, out_hbm,
                idx_v, g0, g1, o0, o1, posv, posrep_v, post_v, sem_g, sem_s):
    info = plsc.get_sparse_core_info()
    num_cores = info.num_cores
    num_workers = num_cores * info.num_subcores
    wid = lax.axis_index("s") * num_cores + lax.axis_index("c")

    batch = patches_hbm.shape[0]
    seqs_per_w = batch // num_workers
    n_groups = seqs_per_w // _GRP
    iota = lax.iota(jnp.int32, _LANES)

    pltpu.sync_copy(pos_hbm, posv)
    pltpu.sync_copy(pos_hbm, posrep_v.at[pl.ds(0, _POS_ROWS)])
    lp0 = posv[_POS_ROWS - 1, pl.ds(0, _LANES)]
    lp1 = posv[_POS_ROWS - 1, pl.ds(_LANES, _LANES)]
    for j in range(_POS_ROWS, _SEQ):
        posrep_v[j, pl.ds(0, _LANES)] = lp0
        posrep_v[j, pl.ds(_LANES, _LANES)] = lp1

    @plsc.parallel_loop(0, _EMBED // 2)
    def post_body(e):
        ecol = jnp.full((_LANES,), e, jnp.int32)
        for s0 in range(_SEQ // _LANES):
            rows = jnp.minimum(iota + (s0 * _LANES), _POS_ROWS - 1)
            post_v[e, pl.ds(s0 * _LANES, _LANES)] = plsc.load_gather(
                posv, [rows, ecol])

    pltpu.sync_copy(patches_hbm.at[pl.ds(wid * seqs_per_w, seqs_per_w)], idx_v)

    gbufs = (g0, g1)
    obufs = (o0, o1)
    ahead = len(gbufs) - 1

    def issue_gathers(g, buf):
        for s in range(_GRP):
            pltpu.async_copy(tok_hbm.at[idx_v.at[g * _GRP + s]],
                             buf.at[pl.ds(s * _SEQ, _SEQ)], sem_g)

    def wait_bytes(buf, sem):
        src = (out_hbm.at[pl.ds(0, buf.shape[0])] if len(buf.shape) == 1
               else tok_hbm.at[pl.ds(0, buf.shape[0])])
        pltpu.make_async_copy(src, buf, sem).wait()

    half = _EMBED // 2
    scat = (iota + half) * _SEQ

    def compute(gc, oc):
        @plsc.parallel_loop(0, half, unroll=2)
        def e_body(k):
            ecol = jnp.full((_LANES,), k, jnp.int32)
            ebase = k * _SEQ
            for s0 in range(_SEQ // _LANES):
                p = post_v[k, pl.ds(s0 * _LANES, _LANES)]
                for s in range(_GRP):
                    base = s * _SEQ + s0 * _LANES
                    v = plsc.load_gather(gc.at[pl.ds(base, _LANES)],
                                         [iota, ecol])
                    oc[pl.ds(ebase + (s * _SEQ * _EMBED + s0 * _LANES),
                             _LANES)] = v + p
            for j in range(_SEQ // _LANES):
                spos = k * (_SEQ // _LANES) + j
                p1 = posrep_v[spos, pl.ds(_LANES, _LANES)]
                for s in range(_GRP):
                    v1 = gc[s * _SEQ + spos, pl.ds(_LANES, _LANES)]
                    plsc.store_scatter(
                        oc, [scat + (s * _SEQ * _EMBED + spos)], v1 + p1)

    for k in range(ahead):
        issue_gathers(k, gbufs[k])

    def h_body(h, _):
        for b in range(len(gbufs)):
            g = h * len(gbufs) + b
            gc, oc = gbufs[b], obufs[b % 2]
            wait_bytes(gc, sem_g)
            @pl.when(g + ahead < n_groups)
            def _():
                issue_gathers(g + ahead, gbufs[(b + ahead) % len(gbufs)])

            @pl.when(g >= 2)
            def _():
                wait_bytes(oc, sem_s)
            compute(gc, oc)
            off = (wid * seqs_per_w + g * _GRP) * (_SEQ * _EMBED)
            pltpu.async_copy(oc, out_hbm.at[pl.ds(off, _CHUNK)], sem_s)
        return 0

    lax.fori_loop(0, n_groups // len(gbufs), h_body, 0)
    wait_bytes(o0, sem_s)
    wait_bytes(o1, sem_s)


def kernel(patches, token_table, pos_table):
    batch, seq = patches.shape
    vocab, embed = token_table.shape
    idx = patches.astype(jnp.int32)


    mesh = plsc.VectorSubcoreMesh(core_axis_name="c", subcore_axis_name="s")
    n_rows = batch * seq

    run = functools.partial(
        pl.kernel,
        out_type=jax.ShapeDtypeStruct((n_rows * embed,), jnp.float32),
        mesh=mesh,
        scratch_types=[
            pltpu.VMEM((batch // 32, seq), jnp.int32),
            pltpu.VMEM((_GRP * seq, embed), jnp.float32),
            pltpu.VMEM((_GRP * seq, embed), jnp.float32),
            pltpu.VMEM((_CHUNK,), jnp.float32),
            pltpu.VMEM((_CHUNK,), jnp.float32),
            pltpu.VMEM((_POS_ROWS, embed), jnp.float32),
            pltpu.VMEM((seq, embed), jnp.float32),
            pltpu.VMEM((embed, seq), jnp.float32),
            pltpu.SemaphoreType.DMA,
            pltpu.SemaphoreType.DMA,
        ],
        compiler_params=pltpu.CompilerParams(use_tc_tiling_on_sc=False,
                                             needs_layout_passes=False),
    )(_emb_kernel)

    out = run(idx, token_table, pos_table)
    return out.reshape(batch, embed, seq).transpose(0, 2, 1)


# speedup vs baseline: 1.0661x; 1.0184x over previous
import functools

import jax
import jax.numpy as jnp
from jax import lax
from jax.experimental import pallas as pl
from jax.experimental.pallas import tpu as pltpu
from jax.experimental.pallas import tpu_sc as plsc

_EMBED = 32
_SEQ = 128
_POS_ROWS = 64
_LANES = 16
_GRP = 4
_CHUNK = _GRP * _SEQ * _EMBED


def _emb_kernel(patches_hbm, tok_hbm, pos_hbm, out_hbm,
                idx_v, g0, g1, o0, o1, posv, posrep_v, post_v, sem_g, sem_s):
    info = plsc.get_sparse_core_info()
    num_cores = info.num_cores
    num_workers = num_cores * info.num_subcores
    wid = lax.axis_index("s") * num_cores + lax.axis_index("c")

    n_groups = patches_hbm.shape[0] // num_workers
    seqs_per_w = n_groups * _GRP
    iota = lax.iota(jnp.int32, _LANES)

    pltpu.sync_copy(pos_hbm, posv)
    pltpu.sync_copy(pos_hbm, posrep_v.at[pl.ds(0, _POS_ROWS)])
    lp0 = posv[_POS_ROWS - 1, pl.ds(0, _LANES)]
    lp1 = posv[_POS_ROWS - 1, pl.ds(_LANES, _LANES)]
    for j in range(_POS_ROWS, _SEQ):
        posrep_v[j, pl.ds(0, _LANES)] = lp0
        posrep_v[j, pl.ds(_LANES, _LANES)] = lp1

    @plsc.parallel_loop(0, _EMBED // 2)
    def post_body(e):
        ecol = jnp.full((_LANES,), e, jnp.int32)
        for s0 in range(_SEQ // _LANES):
            rows = jnp.minimum(iota + (s0 * _LANES), _POS_ROWS - 1)
            post_v[e, pl.ds(s0 * _LANES, _LANES)] = plsc.load_gather(
                posv, [rows, ecol])

    pltpu.sync_copy(patches_hbm.at[pl.ds(wid * n_groups, n_groups)], idx_v)

    gbufs = (g0, g1)
    obufs = (o0, o1)
    ahead = len(gbufs) - 1

    def issue_gathers(g, buf):
        pltpu.async_copy(tok_hbm.at[idx_v.at[g]], buf, sem_g)

    def wait_bytes(buf, sem):
        src = (out_hbm.at[pl.ds(0, buf.shape[0])] if len(buf.shape) == 1
               else tok_hbm.at[pl.ds(0, buf.shape[0])])
        pltpu.make_async_copy(src, buf, sem).wait()

    half = _EMBED // 2
    scat = (iota + half) * _SEQ

    def compute(gc, oc):
        @plsc.parallel_loop(0, half, unroll=2)
        def e_body(k):
            ecol = jnp.full((_LANES,), k, jnp.int32)
            ebase = k * _SEQ
            for s0 in range(_SEQ // _LANES):
                p = post_v[k, pl.ds(s0 * _LANES, _LANES)]
                for s in range(_GRP):
                    base = s * _SEQ + s0 * _LANES
                    v = plsc.load_gather(gc.at[pl.ds(base, _LANES)],
                                         [iota, ecol])
                    oc[pl.ds(ebase + (s * _SEQ * _EMBED + s0 * _LANES),
                             _LANES)] = v + p
            for j in range(_SEQ // _LANES):
                spos = k * (_SEQ // _LANES) + j
                p1 = posrep_v[spos, pl.ds(_LANES, _LANES)]
                for s in range(_GRP):
                    v1 = gc[s * _SEQ + spos, pl.ds(_LANES, _LANES)]
                    plsc.store_scatter(
                        oc, [scat + (s * _SEQ * _EMBED + spos)], v1 + p1)

    for k in range(ahead):
        issue_gathers(k, gbufs[k])

    def h_body(h, _):
        for b in range(len(gbufs)):
            g = h * len(gbufs) + b
            gc, oc = gbufs[b], obufs[b % 2]
            wait_bytes(gc, sem_g)
            @pl.when(g + ahead < n_groups)
            def _():
                issue_gathers(g + ahead, gbufs[(b + ahead) % len(gbufs)])

            @pl.when(g >= 2)
            def _():
                wait_bytes(oc, sem_s)
            compute(gc, oc)
            off = (wid * seqs_per_w + g * _GRP) * (_SEQ * _EMBED)
            pltpu.async_copy(oc, out_hbm.at[pl.ds(off, _CHUNK)], sem_s)
        return 0

    lax.fori_loop(0, n_groups // len(gbufs), h_body, 0)
    wait_bytes(o0, sem_s)
    wait_bytes(o1, sem_s)


def kernel(patches, token_table, pos_table):
    batch, seq = patches.shape
    vocab, embed = token_table.shape
    idx = patches.astype(jnp.int32)


    mesh = plsc.VectorSubcoreMesh(core_axis_name="c", subcore_axis_name="s")
    n_rows = batch * seq

    run = functools.partial(
        pl.kernel,
        out_type=jax.ShapeDtypeStruct((n_rows * embed,), jnp.float32),
        mesh=mesh,
        scratch_types=[
            pltpu.VMEM((batch * seq // (32 * _GRP * seq), _GRP * seq),
                       jnp.int32),
            pltpu.VMEM((_GRP * seq, embed), jnp.float32),
            pltpu.VMEM((_GRP * seq, embed), jnp.float32),
            pltpu.VMEM((_CHUNK,), jnp.float32),
            pltpu.VMEM((_CHUNK,), jnp.float32),
            pltpu.VMEM((_POS_ROWS, embed), jnp.float32),
            pltpu.VMEM((seq, embed), jnp.float32),
            pltpu.VMEM((embed, seq), jnp.float32),
            pltpu.SemaphoreType.DMA,
            pltpu.SemaphoreType.DMA,
        ],
        compiler_params=pltpu.CompilerParams(use_tc_tiling_on_sc=False,
                                             needs_layout_passes=False),
    )(_emb_kernel)

    out = run(idx.reshape(-1, _GRP * seq), token_table, pos_table)
    return out.reshape(batch, embed, seq).transpose(0, 2, 1)
